# baseline (speedup 1.0000x reference)
import functools

import jax
import jax.numpy as jnp
from jax import lax
from jax.experimental import pallas as pl
from jax.experimental.pallas import tpu as pltpu
from jax.experimental.pallas import tpu_sc as plsc

NW = 32


def _pad16(k: int) -> int:
    return ((k + 15) // 16) * 16


def _build_list(dst, src, beta, seq, valid, nw_shift):
    m = dst.shape[0]
    big = jnp.int32(2**30)
    key = jnp.where(valid, dst * 2 + seq, big)
    order = jnp.argsort(key)
    dst_s = dst[order]
    src_s = src[order]
    beta_s = beta[order]
    cnt = jnp.sum(valid.astype(jnp.int32))
    ks = jnp.arange(m, dtype=jnp.int32)
    in_range = ks < cnt
    first = jnp.concatenate([jnp.ones((1,), jnp.bool_), dst_s[1:] != dst_s[:-1]])
    last = jnp.concatenate([dst_s[1:] != dst_s[:-1], jnp.ones((1,), jnp.bool_)])
    mul = jnp.where(first, 0.0, 1.0).astype(jnp.float32)
    last_i = last.astype(jnp.int32)
    wkey = jnp.where(in_range, dst_s >> nw_shift, jnp.int32(NW))
    offs = jnp.searchsorted(wkey, jnp.arange(NW + 1, dtype=jnp.int32)).astype(jnp.int32)
    npad = _pad16(m)
    pad = npad - m

    def p(a):
        return jnp.pad(a, (0, pad + 16))

    offs = jnp.pad(offs, (0, 48 - (NW + 1)))
    return p(dst_s), p(src_s), p(beta_s), p(mul), p(last_i), offs


def _build_meta(ref_index, target_index, mag):
    n = ref_index.shape[0]
    i_ar = jnp.arange(n, dtype=jnp.int32)
    r = ref_index.astype(jnp.int32)
    t = target_index.astype(jnp.int32)

    p = jnp.argsort(r * 4096 + i_ar)
    r_s = r[p]
    is_last_s = jnp.concatenate([r_s[1:] != r_s[:-1], jnp.ones((1,), jnp.bool_)])
    is_winner = jnp.zeros((n,), jnp.bool_).at[p].set(is_last_s)

    q = jnp.argsort(t * 4096 + i_ar)
    t_q = t[q]
    is_first_q = jnp.concatenate([jnp.ones((1,), jnp.bool_), t_q[1:] != t_q[:-1]])
    is_first_tgt = jnp.zeros((n,), jnp.bool_).at[q].set(is_first_q)

    r_sorted = jnp.sort(r)
    pos = jnp.clip(jnp.searchsorted(r_sorted, t), 0, n - 1)
    t_in_ref = r_sorted[pos] == t

    am = jnp.abs(mag)
    a1m = jnp.abs(1.0 - mag)
    tot = am + a1m
    tp = am / tot
    rp = a1m / tot

    ones = jnp.ones((n,), jnp.float32)
    zeros_i = jnp.zeros((n,), jnp.int32)
    ones_i = jnp.ones((n,), jnp.int32)

    y_dst = jnp.concatenate([r, t, t])
    y_src = jnp.concatenate([r, t, r])
    y_beta = jnp.concatenate([rp, ones, tp])
    y_seq = jnp.concatenate([zeros_i, zeros_i, ones_i])
    y_valid = jnp.concatenate(
        [is_winner, is_first_tgt & ~t_in_ref, ~t_in_ref])
    y_meta = _build_list(y_dst, y_src, y_beta, y_seq, y_valid, 9)

    x_dst = jnp.concatenate([r, r])
    x_src = jnp.concatenate([t, r])
    x_beta = jnp.concatenate([mag, 1.0 - mag])
    x_seq = jnp.concatenate([zeros_i, ones_i])
    x_valid = jnp.concatenate([is_winner, is_winner])
    x_meta = _build_list(x_dst, x_src, x_beta, x_seq, x_valid, 9)
    return y_meta, x_meta


def _worker_process(dst_v, src_v, beta_v, mul_v, last_v, lo, hi, src_hbm,
                    out_hbm, tmp, acc, width, width_pad):
    def item(k, carry):
        d = dst_v[pl.ds(k, 16)][0]
        s = src_v[pl.ds(k, 16)][0]
        b = beta_v[pl.ds(k, 16)][0]
        mm = mul_v[pl.ds(k, 16)][0]
        lf = last_v[pl.ds(k, 16)][0]
        pltpu.sync_copy(src_hbm.at[s], tmp.at[pl.ds(0, width)])

        @pl.loop(0, width_pad, step=16)
        def _(c):
            acc[pl.ds(c, 16)] = acc[pl.ds(c, 16)] * mm + tmp[pl.ds(c, 16)] * b

        @pl.when(lf == 1)
        def _():
            pltpu.sync_copy(acc.at[pl.ds(0, width)], out_hbm.at[d])

        return carry

    lax.fori_loop(lo, hi, item, 0)


def _sc_kernel_body(B, D, C, C_PAD, NPY, NPX, ROWS_W, x_hbm, y_hbm,
                    ydst_h, ysrc_h, ybet_h, ymul_h, ylst_h, yoff_h,
                    xdst_h, xsrc_h, xbet_h, xmul_h, xlst_h, xoff_h,
                    ret_hbm, rety_hbm,
                    ydst_v, ysrc_v, ybet_v, ymul_v, ylst_v, yoff_v,
                    xdst_v, xsrc_v, xbet_v, xmul_v, xlst_v, xoff_v,
                    tmp_y, acc_y, tmp_x, acc_x, cbuf_y, cbuf_x):
    w = lax.axis_index("c") * 16 + lax.axis_index("s")
    row0 = w * ROWS_W

    for hh, vv in ((ydst_h, ydst_v), (ysrc_h, ysrc_v), (ybet_h, ybet_v),
                   (ymul_h, ymul_v), (ylst_h, ylst_v), (yoff_h, yoff_v),
                   (xdst_h, xdst_v), (xsrc_h, xsrc_v), (xbet_h, xbet_v),
                   (xmul_h, xmul_v), (xlst_h, xlst_v), (xoff_h, xoff_v)):
        pltpu.sync_copy(hh, vv)

    CHY = 16
    CHX = 128

    @pl.loop(0, ROWS_W, step=CHY)
    def _(bb):
        pltpu.sync_copy(y_hbm.at[pl.ds(row0 + bb, CHY)], cbuf_y)
        pltpu.sync_copy(cbuf_y, rety_hbm.at[pl.ds(row0 + bb, CHY)])

    @pl.loop(0, ROWS_W, step=CHX)
    def _(bb):
        pltpu.sync_copy(x_hbm.at[pl.ds(row0 + bb, CHX)], cbuf_x)
        pltpu.sync_copy(cbuf_x, ret_hbm.at[pl.ds(row0 + bb, CHX)])

    @pl.loop(0, C_PAD, step=16)
    def _(c):
        acc_y[pl.ds(c, 16)] = jnp.zeros((16,), jnp.float32)
        tmp_y[pl.ds(c, 16)] = jnp.zeros((16,), jnp.float32)

    @pl.loop(0, D, step=16)
    def _(c):
        acc_x[pl.ds(c, 16)] = jnp.zeros((16,), jnp.float32)

    yoff16 = yoff_v[pl.ds(w, 16)]
    xoff16 = xoff_v[pl.ds(w, 16)]
    _worker_process(ydst_v, ysrc_v, ybet_v, ymul_v, ylst_v,
                    yoff16[0], yoff16[1], y_hbm, rety_hbm,
                    tmp_y, acc_y, C, C_PAD)
    _worker_process(xdst_v, xsrc_v, xbet_v, xmul_v, xlst_v,
                    xoff16[0], xoff16[1], x_hbm, ret_hbm,
                    tmp_x, acc_x, D, D)


@jax.jit
def kernel(x, y, ref_index, target_index, mag):
    B, D = x.shape
    C = y.shape[1]
    n = ref_index.shape[0]
    C_PAD = _pad16(C)
    NPY = _pad16(3 * n) + 16
    NPX = _pad16(2 * n) + 16
    ROWS_W = B // NW

    y_meta, x_meta = _build_meta(ref_index, target_index, mag)

    mesh = plsc.VectorSubcoreMesh(core_axis_name="c", subcore_axis_name="s")
    body = functools.partial(_sc_kernel_body, B, D, C, C_PAD, NPY, NPX, ROWS_W)
    f = pl.kernel(
        body,
        out_type=(jax.ShapeDtypeStruct((B, D), jnp.float32),
                  jax.ShapeDtypeStruct((B, C), jnp.float32)),
        mesh=mesh,
        scratch_types=[
            pltpu.VMEM((NPY,), jnp.int32),
            pltpu.VMEM((NPY,), jnp.int32),
            pltpu.VMEM((NPY,), jnp.float32),
            pltpu.VMEM((NPY,), jnp.float32),
            pltpu.VMEM((NPY,), jnp.int32),
            pltpu.VMEM((48,), jnp.int32),
            pltpu.VMEM((NPX,), jnp.int32),
            pltpu.VMEM((NPX,), jnp.int32),
            pltpu.VMEM((NPX,), jnp.float32),
            pltpu.VMEM((NPX,), jnp.float32),
            pltpu.VMEM((NPX,), jnp.int32),
            pltpu.VMEM((48,), jnp.int32),
            pltpu.VMEM((C_PAD,), jnp.float32),
            pltpu.VMEM((C_PAD,), jnp.float32),
            pltpu.VMEM((D,), jnp.float32),
            pltpu.VMEM((D,), jnp.float32),
            pltpu.VMEM((16, C), jnp.float32),
            pltpu.VMEM((128, D), jnp.float32),
        ],
        compiler_params=pltpu.CompilerParams(use_tc_tiling_on_sc=False),
    )
    ret, ret_y = f(x, y, *y_meta, *x_meta)
    return (ret, ret_y)
